# baseline (speedup 1.0000x reference)
import jax
import jax.numpy as jnp
from jax.experimental import pallas as pl


def _xw_kernel(x_ref, w_ref, o_ref):
    o_ref[...] = jnp.dot(x_ref[...], w_ref[...],
                         preferred_element_type=jnp.float32)


def _layer1_kernel(adj_ref, s1_ref, b1_ref, w2_ref, s2_ref):
    h = jnp.dot(adj_ref[...], s1_ref[...],
                preferred_element_type=jnp.float32)
    h = jnp.maximum(h + b1_ref[...], 0.0)
    s2_ref[...] = jnp.dot(h, w2_ref[...],
                          preferred_element_type=jnp.float32)


def _layer2_kernel(adj_ref, s2_ref, b2_ref, o_ref):
    o = jnp.dot(adj_ref[...], s2_ref[...],
                preferred_element_type=jnp.float32)
    o = o + b2_ref[...]
    m = jnp.max(o, axis=-1, keepdims=True)
    e = o - m
    lse = jnp.log(jnp.sum(jnp.exp(e), axis=-1, keepdims=True))
    o_ref[...] = e - lse


def kernel(x, adj, W1, b1, W2, b2):
    n, _ = x.shape
    hid = W1.shape[1]
    out_f = W2.shape[1]
    bm = 400

    s1 = pl.pallas_call(
        _xw_kernel,
        out_shape=jax.ShapeDtypeStruct((n, hid), jnp.float32),
    )(x, W1)

    b1r = b1.reshape(1, hid)
    b2r = b2.reshape(1, out_f)
    grid = (n // bm,)

    s2 = pl.pallas_call(
        _layer1_kernel,
        grid=grid,
        in_specs=[
            pl.BlockSpec((bm, n), lambda i: (i, 0)),
            pl.BlockSpec((n, hid), lambda i: (0, 0)),
            pl.BlockSpec((1, hid), lambda i: (0, 0)),
            pl.BlockSpec((hid, out_f), lambda i: (0, 0)),
        ],
        out_specs=pl.BlockSpec((bm, out_f), lambda i: (i, 0)),
        out_shape=jax.ShapeDtypeStruct((n, out_f), jnp.float32),
    )(adj, s1, b1r, W2)

    out = pl.pallas_call(
        _layer2_kernel,
        grid=grid,
        in_specs=[
            pl.BlockSpec((bm, n), lambda i: (i, 0)),
            pl.BlockSpec((n, out_f), lambda i: (0, 0)),
            pl.BlockSpec((1, out_f), lambda i: (0, 0)),
        ],
        out_specs=pl.BlockSpec((bm, out_f), lambda i: (i, 0)),
        out_shape=jax.ShapeDtypeStruct((n, out_f), jnp.float32),
    )(adj, s2, b2r)
    return out


# speedup vs baseline: 1.0130x; 1.0130x over previous
import jax
import jax.numpy as jnp
from jax.experimental import pallas as pl
from jax.experimental.pallas import tpu as pltpu

_BM = 400


def _xw_kernel(x_ref, w_ref, o_ref):
    o_ref[...] = jnp.dot(x_ref[...], w_ref[...],
                         preferred_element_type=jnp.float32)


def _fused_kernel(adj_ref, s1_ref, b1_ref, w2_ref, b2_ref, o_ref, s2_ref):
    p = pl.program_id(0)
    i = pl.program_id(1)

    @pl.when(p == 0)
    def _phase0():
        h = jnp.dot(adj_ref[...], s1_ref[...],
                    preferred_element_type=jnp.float32)
        h = jnp.maximum(h + b1_ref[...], 0.0)
        s2_blk = jnp.dot(h, w2_ref[...], preferred_element_type=jnp.float32)
        s2_ref[pl.ds(i * _BM, _BM), :] = s2_blk
        o_ref[0] = s2_blk

    @pl.when(p == 1)
    def _phase1():
        o = jnp.dot(adj_ref[...], s2_ref[...],
                    preferred_element_type=jnp.float32)
        o = o + b2_ref[...]
        m = jnp.max(o, axis=-1, keepdims=True)
        e = o - m
        lse = jnp.log(jnp.sum(jnp.exp(e), axis=-1, keepdims=True))
        o_ref[0] = e - lse


def kernel(x, adj, W1, b1, W2, b2):
    n, _ = x.shape
    hid = W1.shape[1]
    out_f = W2.shape[1]

    s1 = pl.pallas_call(
        _xw_kernel,
        out_shape=jax.ShapeDtypeStruct((n, hid), jnp.float32),
    )(x, W1)

    b1r = b1.reshape(1, hid)
    b2r = b2.reshape(1, out_f)

    out = pl.pallas_call(
        _fused_kernel,
        grid=(2, n // _BM),
        in_specs=[
            pl.BlockSpec((_BM, n), lambda p, i: (i, 0)),
            pl.BlockSpec((n, hid), lambda p, i: (0, 0)),
            pl.BlockSpec((1, hid), lambda p, i: (0, 0)),
            pl.BlockSpec((hid, out_f), lambda p, i: (0, 0)),
            pl.BlockSpec((1, out_f), lambda p, i: (0, 0)),
        ],
        out_specs=pl.BlockSpec((1, _BM, out_f), lambda p, i: (p, i, 0)),
        out_shape=jax.ShapeDtypeStruct((2, n, out_f), jnp.float32),
        scratch_shapes=[pltpu.VMEM((n, out_f), jnp.float32)],
    )(adj, s1, b1r, W2, b2r)
    return out[1]
